# Initial kernel scaffold; baseline (speedup 1.0000x reference)
import jax
import jax.numpy as jnp
from jax.experimental import pallas as pl


def kernel(nodes, edges, types, table, W, root, bias, gprompt, in_proj_w, in_proj_b, out_proj_w, out_proj_b):
    raise NotImplementedError("write your pallas kernel here")


# speedup vs baseline: 2.8054x; 2.8054x over previous
import functools

import jax
import jax.numpy as jnp
from jax import lax
from jax.experimental import pallas as pl
from jax.experimental.pallas import tpu as pltpu
from jax.experimental.pallas import tpu_sc as plsc

_LG = 16
_NC = 2
_NS = 16
_NW = _NC * _NS


@functools.lru_cache(maxsize=None)
def _gather_rows(n_table, n_rows, d):
  ch = 64
  rpw = n_rows // _NW
  n_ch = rpw // ch
  mesh = plsc.VectorSubcoreMesh(core_axis_name="c", subcore_axis_name="s")

  @functools.partial(
      pl.kernel,
      out_type=jax.ShapeDtypeStruct((n_rows, d), jnp.float32),
      mesh=mesh,
      scratch_types=[
          pltpu.VMEM((n_ch, ch), jnp.int32),
          pltpu.VMEM((ch, d), jnp.float32),
          pltpu.SemaphoreType.DMA,
      ],
  )
  def k(table_hbm, idx_hbm, out_hbm, idx_v, rows_v, sem):
    cid = lax.axis_index("c")
    sid = lax.axis_index("s")
    wid = sid * _NC + cid
    base = wid * rpw
    pltpu.sync_copy(idx_hbm.at[wid], idx_v)

    def body(c, carry):
      pltpu.async_copy(table_hbm.at[idx_v.at[c]], rows_v, sem).wait()
      pltpu.sync_copy(rows_v, out_hbm.at[pl.ds(base + c * ch, ch)])
      return carry

    lax.fori_loop(0, n_ch, body, 0)

  return k


@functools.lru_cache(maxsize=None)
def _segsum(npg, e, r, d, with_cnt):
  chn = d // _LG
  nchc = chn // _NC
  ept = e // _NS
  g = ept // 128
  arows = r * npg
  trows = arows // _NS
  cnr = arows // _LG
  crt = cnr // _NS
  mesh = plsc.VectorSubcoreMesh(core_axis_name="c", subcore_axis_name="s")

  out_type = [jax.ShapeDtypeStruct((arows, chn, _LG), jnp.float32)]
  if with_cnt:
    out_type.append(jax.ShapeDtypeStruct((arows, _LG), jnp.float32))

  scratch = [
      pltpu.VMEM((g, 128), jnp.int32),
      pltpu.VMEM((g, 128), jnp.int32),
      pltpu.VMEM((g, 128), jnp.int32),
      pltpu.VMEM((128, _LG), jnp.float32),
      pltpu.VMEM((512, _LG), jnp.float32),
      pltpu.SemaphoreType.DMA,
      pltpu.VMEM_SHARED((arows, _LG), jnp.float32),
  ]

  def body(*refs):
    if with_cnt:
      (x_hbm, src_hbm, dst_hbm, et_hbm, sum_hbm, cnt_hbm,
       srcC, fdst, idxv, rows, zbuf, sem, acc) = refs
    else:
      (x_hbm, src_hbm, dst_hbm, et_hbm, sum_hbm,
       srcC, fdst, idxv, rows, zbuf, sem, acc) = refs

    cid = lax.axis_index("c")
    sid = lax.axis_index("s")

    pltpu.sync_copy(src_hbm.at[sid], srcC)
    pltpu.sync_copy(dst_hbm.at[sid], fdst)
    pltpu.sync_copy(et_hbm.at[sid], idxv)

    zv = jnp.zeros((_LG,), jnp.float32)

    def zfill(i, carry):
      zbuf[i, :] = zv
      return carry

    lax.fori_loop(0, 512, zfill, 0)

    def fixup(i, carry):
      j = i // 8
      kk = i % 8
      sl = pl.ds(kk * _LG, _LG)
      fdst[j, sl] = idxv[j, sl] * npg + fdst[j, sl]
      srcC[j, sl] = srcC[j, sl] * chn
      return carry

    lax.fori_loop(0, g * 8, fixup, 0)
    plsc.subcore_barrier()

    if with_cnt:
      @pl.when(cid == 0)
      def _count():
        for z in range(trows // 512):
          pltpu.sync_copy(zbuf, acc.at[pl.ds(sid * trows + z * 512, 512)])
        ones = jnp.ones((_LG,), jnp.float32)

        def ofill(i, carry):
          rows[i, :] = ones
          return carry

        lax.fori_loop(0, 128, ofill, 0)
        plsc.subcore_barrier()

        def cb(j, carry):
          pltpu.sync_copy(rows, acc.at[fdst.at[j]], add=True)
          return carry

        lax.fori_loop(0, g, cb, 0)
        plsc.subcore_barrier()
        pltpu.sync_copy(acc.at[pl.ds(sid * trows, trows)],
                        cnt_hbm.at[pl.ds(sid * trows, trows)])

    def chunk(c, carry):
      cc = cid * nchc + c
      def ib(i, cy2):
        j = i // 8
        kk = i % 8
        sl = pl.ds(kk * _LG, _LG)
        idxv[j, sl] = srcC[j, sl] + cc
        return cy2

      lax.fori_loop(0, g * 8, ib, 0)
      for z in range(trows // 512):
        pltpu.sync_copy(zbuf, acc.at[pl.ds(sid * trows + z * 512, 512)])
      plsc.subcore_barrier()

      def grp(j, cy):
        pltpu.async_copy(x_hbm.at[idxv.at[j]], rows, sem).wait()
        pltpu.sync_copy(rows, acc.at[fdst.at[j]], add=True)
        return cy

      lax.fori_loop(0, g, grp, 0)
      plsc.subcore_barrier()
      pltpu.sync_copy(acc.at[pl.ds(sid * trows, trows)],
                      sum_hbm.at[pl.ds(sid * trows, trows), cc])
      return carry

    lax.fori_loop(0, nchc, chunk, 0)

  return pl.kernel(
      body,
      out_type=tuple(out_type) if with_cnt else out_type[0],
      mesh=mesh,
      scratch_types=scratch,
      compiler_params=pltpu.CompilerParams(use_tc_tiling_on_sc=False),
  )


@functools.lru_cache(maxsize=None)
def _combine(npg, d, r, relu, bm=256):
  def body(x_ref, sum_ref, cnt_ref, w_ref, root_ref, bias_ref, o_ref):
    x = x_ref[...]
    inv = 1.0 / jnp.maximum(cnt_ref[...], 1.0)
    acc = jnp.dot(x, root_ref[...], preferred_element_type=jnp.float32)
    acc = acc + bias_ref[...]
    for rr in range(r):
      m = sum_ref[rr] * inv[:, rr:rr + 1]
      acc = acc + jnp.dot(m, w_ref[rr], preferred_element_type=jnp.float32)
    if relu:
      acc = jnp.maximum(acc, 0.0)
    o_ref[...] = acc

  return pl.pallas_call(
      body,
      grid=(npg // bm,),
      in_specs=[
          pl.BlockSpec((bm, d), lambda i: (i, 0)),
          pl.BlockSpec((r, bm, d), lambda i: (0, i, 0)),
          pl.BlockSpec((bm, r), lambda i: (i, 0)),
          pl.BlockSpec((r, d, d), lambda i: (0, 0, 0)),
          pl.BlockSpec((d, d), lambda i: (0, 0)),
          pl.BlockSpec((1, d), lambda i: (0, 0)),
      ],
      out_specs=pl.BlockSpec((bm, d), lambda i: (i, 0)),
      out_shape=jax.ShapeDtypeStruct((npg, d), jnp.float32),
  )


@functools.lru_cache(maxsize=None)
def _mha(b, npg, d, h, kb=512):
  hd = d // h
  nkv = npg // kb
  p16 = 16
  scale = 1.0 / (hd ** 0.5)

  def body(q_ref, x_ref, wq_ref, wk_ref, wv_ref, bq_ref, bk_ref, bv_ref,
           wo_ref, bo_ref, o_ref, qh_ref, s_ref, acc_ref):
    j = pl.program_id(1)

    @pl.when(j == 0)
    def _():
      qh_ref[...] = (jnp.dot(q_ref[...], wq_ref[...],
                             preferred_element_type=jnp.float32)
                     + bq_ref[...])

    @pl.when(j < nkv)
    def _scores():
      x = x_ref[0]
      k = jnp.dot(x, wk_ref[...], preferred_element_type=jnp.float32) \
          + bk_ref[...]
      jj = j
      for hh in range(h):
        qh = qh_ref[:, hh * hd:(hh + 1) * hd]
        kh = k[:, hh * hd:(hh + 1) * hd]
        s = lax.dot_general(qh, kh, (((1,), (1,)), ((), ())),
                            preferred_element_type=jnp.float32) * scale
        s_ref[hh, :, pl.ds(jj * kb, kb)] = s

    @pl.when(j == nkv)
    def _softmax():
      s = s_ref[...]
      m = jnp.max(s, axis=-1, keepdims=True)
      p = jnp.exp(s - m)
      s_ref[...] = p / jnp.sum(p, axis=-1, keepdims=True)
      acc_ref[...] = jnp.zeros_like(acc_ref)

    @pl.when(j >= nkv)
    def _values():
      x = x_ref[0]
      v = jnp.dot(x, wv_ref[...], preferred_element_type=jnp.float32) \
          + bv_ref[...]
      jj = j - nkv
      for hh in range(h):
        ph = s_ref[hh, :, pl.ds(jj * kb, kb)]
        vh = v[:, hh * hd:(hh + 1) * hd]
        acc_ref[hh] = acc_ref[hh] + jnp.dot(
            ph, vh, preferred_element_type=jnp.float32)

    @pl.when(j == 2 * nkv - 1)
    def _out():
      o = bo_ref[...] + jnp.zeros((p16, d), jnp.float32)
      for hh in range(h):
        o = o + jnp.dot(acc_ref[hh], wo_ref[pl.ds(hh * hd, hd)],
                        preferred_element_type=jnp.float32)
      o_ref[0] = o

  return pl.pallas_call(
      body,
      grid=(b, 2 * nkv),
      in_specs=[
          pl.BlockSpec((p16, d), lambda bb, j: (0, 0)),
          pl.BlockSpec((1, kb, d), lambda bb, j: (bb, j % nkv, 0)),
          pl.BlockSpec((d, d), lambda bb, j: (0, 0)),
          pl.BlockSpec((d, d), lambda bb, j: (0, 0)),
          pl.BlockSpec((d, d), lambda bb, j: (0, 0)),
          pl.BlockSpec((1, d), lambda bb, j: (0, 0)),
          pl.BlockSpec((1, d), lambda bb, j: (0, 0)),
          pl.BlockSpec((1, d), lambda bb, j: (0, 0)),
          pl.BlockSpec((d, d), lambda bb, j: (0, 0)),
          pl.BlockSpec((1, d), lambda bb, j: (0, 0)),
      ],
      out_specs=pl.BlockSpec((1, p16, d), lambda bb, j: (bb, 0, 0)),
      out_shape=jax.ShapeDtypeStruct((b, p16, d), jnp.float32),
      scratch_shapes=[
          pltpu.VMEM((p16, d), jnp.float32),
          pltpu.VMEM((h, p16, npg), jnp.float32),
          pltpu.VMEM((h, p16, hd), jnp.float32),
      ],
  )


def kernel(nodes, edges, types, table, W, root, bias, gprompt,
           in_proj_w, in_proj_b, out_proj_w, out_proj_b):
  b, npg = nodes.shape
  e = edges.shape[2]
  n_table, d = table.shape
  n_layers, r = W.shape[0], W.shape[1]
  p = gprompt.shape[0]
  h = 8
  chn = d // _LG

  nrows = b * npg
  idx3 = nodes.reshape(_NW, nrows // _NW // 64, 64)
  x0 = _gather_rows(n_table, nrows, d)(table, idx3).reshape(b, npg, d)

  def tile3(a):
    return a.reshape(_NS, e // _NS // 128, 128).astype(jnp.int32)

  embs = []
  for bb in range(b):
    src3 = tile3(edges[bb, 0])
    dst3 = tile3(edges[bb, 1])
    et3 = tile3(types[bb])
    x = x0[bb]
    cnt = None
    for l in range(n_layers):
      xv = x.reshape(npg * chn, _LG)
      if l == 0:
        sum3, cnt2 = _segsum(npg, e, r, d, True)(xv, src3, dst3, et3)
        cnt = cnt2[:, 0].reshape(r, npg).T
      else:
        sum3 = _segsum(npg, e, r, d, False)(xv, src3, dst3, et3)
      summed = sum3.reshape(r, npg, d)
      x = _combine(npg, d, r, l < n_layers - 1)(
          x, summed, cnt, W[l], root[l], bias[l][None])
    embs.append(x)

  node_embeddings = jnp.stack(embs, 0)

  gp16 = jnp.zeros((16, d), jnp.float32).at[:p].set(gprompt)
  wq_t = in_proj_w[:d].T
  wk_t = in_proj_w[d:2 * d].T
  wv_t = in_proj_w[2 * d:].T
  bq = in_proj_b[:d][None]
  bk = in_proj_b[d:2 * d][None]
  bv = in_proj_b[2 * d:][None]
  wo_t = out_proj_w.T
  bo = out_proj_b[None]
  mha_out = _mha(b, npg, d, h)(gp16, node_embeddings, wq_t, wk_t, wv_t,
                               bq, bk, bv, wo_t, bo)
  agg_embeddings = mha_out[:, :p, :]
  return (node_embeddings, agg_embeddings)


# speedup vs baseline: 3.8920x; 1.3873x over previous
import functools

import jax
import jax.numpy as jnp
from jax import lax
from jax.experimental import pallas as pl
from jax.experimental.pallas import tpu as pltpu
from jax.experimental.pallas import tpu_sc as plsc

_LG = 16
_NC = 2
_NS = 16
_NW = _NC * _NS


@functools.lru_cache(maxsize=None)
def _gather_rows(n_table, n_rows, d):
  ch = 64
  rpw = n_rows // _NW
  n_ch = rpw // ch
  mesh = plsc.VectorSubcoreMesh(core_axis_name="c", subcore_axis_name="s")

  @functools.partial(
      pl.kernel,
      out_type=jax.ShapeDtypeStruct((n_rows, d), jnp.float32),
      mesh=mesh,
      scratch_types=[
          pltpu.VMEM((n_ch, ch), jnp.int32),
          pltpu.VMEM((ch, d), jnp.float32),
          pltpu.SemaphoreType.DMA,
      ],
  )
  def k(table_hbm, idx_hbm, out_hbm, idx_v, rows_v, sem):
    cid = lax.axis_index("c")
    sid = lax.axis_index("s")
    wid = sid * _NC + cid
    base = wid * rpw
    pltpu.sync_copy(idx_hbm.at[wid], idx_v)

    def body(c, carry):
      pltpu.async_copy(table_hbm.at[idx_v.at[c]], rows_v, sem).wait()
      pltpu.sync_copy(rows_v, out_hbm.at[pl.ds(base + c * ch, ch)])
      return carry

    lax.fori_loop(0, n_ch, body, 0)

  return k


@functools.lru_cache(maxsize=None)
def _segsum(npg, e, r, d, with_cnt):
  chn = d // _LG
  nchc = chn // _NC
  ept = e // _NS
  g = ept // 128
  arows = r * npg
  trows = arows // _NS
  cnr = arows // _LG
  crt = cnr // _NS
  mesh = plsc.VectorSubcoreMesh(core_axis_name="c", subcore_axis_name="s")

  out_type = [jax.ShapeDtypeStruct((arows, chn, _LG), jnp.float32)]
  if with_cnt:
    out_type.append(jax.ShapeDtypeStruct((arows, _LG), jnp.float32))

  scratch = [
      pltpu.VMEM((g, 128), jnp.int32),
      pltpu.VMEM((g, 128), jnp.int32),
      pltpu.VMEM((g, 128), jnp.int32),
      pltpu.VMEM((16, 128, _LG), jnp.float32),
      pltpu.VMEM((512, _LG), jnp.float32),
      pltpu.SemaphoreType.DMA,
      pltpu.SemaphoreType.DMA,
      pltpu.VMEM_SHARED((arows, _LG), jnp.float32),
  ]

  def body(*refs):
    if with_cnt:
      (x_hbm, src_hbm, dst_hbm, et_hbm, sum_hbm, cnt_hbm,
       srcC, fdst, idxv, rows, zbuf, sem, sem2, acc) = refs
    else:
      (x_hbm, src_hbm, dst_hbm, et_hbm, sum_hbm,
       srcC, fdst, idxv, rows, zbuf, sem, sem2, acc) = refs

    cid = lax.axis_index("c")
    sid = lax.axis_index("s")

    pltpu.sync_copy(src_hbm.at[sid], srcC)
    pltpu.sync_copy(dst_hbm.at[sid], fdst)
    pltpu.sync_copy(et_hbm.at[sid], idxv)

    zv = jnp.zeros((_LG,), jnp.float32)

    def zfill(i, carry):
      zbuf[i, :] = zv
      return carry

    lax.fori_loop(0, 512, zfill, 0)

    def fixup(i, carry):
      j = i // 8
      kk = i % 8
      sl = pl.ds(kk * _LG, _LG)
      fdst[j, sl] = idxv[j, sl] * npg + fdst[j, sl]
      srcC[j, sl] = srcC[j, sl] * chn
      return carry

    lax.fori_loop(0, g * 8, fixup, 0)
    plsc.subcore_barrier()

    if with_cnt:
      @pl.when(cid == 0)
      def _count():
        for z in range(trows // 512):
          pltpu.sync_copy(zbuf, acc.at[pl.ds(sid * trows + z * 512, 512)])
        ones = jnp.ones((_LG,), jnp.float32)

        def ofill(i, carry):
          rows[0, i, :] = ones
          return carry

        lax.fori_loop(0, 128, ofill, 0)
        plsc.subcore_barrier()

        def cb(j, carry):
          pltpu.sync_copy(rows.at[0], acc.at[fdst.at[j]], add=True)
          return carry

        lax.fori_loop(0, g, cb, 0)
        plsc.subcore_barrier()
        pltpu.sync_copy(acc.at[pl.ds(sid * trows, trows)],
                        cnt_hbm.at[pl.ds(sid * trows, trows)])

    def chunk(c, carry):
      cc = cid * nchc + c
      def ib(i, cy2):
        j = i // 8
        kk = i % 8
        sl = pl.ds(kk * _LG, _LG)
        idxv[j, sl] = srcC[j, sl] + cc
        return cy2

      lax.fori_loop(0, g * 8, ib, 0)
      for z in range(trows // 512):
        pltpu.sync_copy(zbuf, acc.at[pl.ds(sid * trows + z * 512, 512)])
      plsc.subcore_barrier()

      def sb_loop(sb, cy):
        base = sb * 16
        gs = [pltpu.async_copy(x_hbm.at[idxv.at[base + j]],
                               rows.at[j], sem) for j in range(16)]
        for dd in gs:
          dd.wait()
        ss = [pltpu.async_copy(rows.at[j], acc.at[fdst.at[base + j]],
                               sem2, add=True) for j in range(16)]
        for dd in ss:
          dd.wait()
        return cy

      lax.fori_loop(0, g // 16, sb_loop, 0)
      plsc.subcore_barrier()
      pltpu.sync_copy(acc.at[pl.ds(sid * trows, trows)],
                      sum_hbm.at[pl.ds(sid * trows, trows), cc])
      return carry

    lax.fori_loop(0, nchc, chunk, 0)

  return pl.kernel(
      body,
      out_type=tuple(out_type) if with_cnt else out_type[0],
      mesh=mesh,
      scratch_types=scratch,
      compiler_params=pltpu.CompilerParams(use_tc_tiling_on_sc=False),
  )


@functools.lru_cache(maxsize=None)
def _combine(npg, d, r, relu, bm=256):
  def body(x_ref, sum_ref, cnt_ref, w_ref, root_ref, bias_ref, o_ref):
    x = x_ref[...]
    inv = 1.0 / jnp.maximum(cnt_ref[...], 1.0)
    acc = jnp.dot(x, root_ref[...], preferred_element_type=jnp.float32)
    acc = acc + bias_ref[...]
    for rr in range(r):
      m = sum_ref[rr] * inv[:, rr:rr + 1]
      acc = acc + jnp.dot(m, w_ref[rr], preferred_element_type=jnp.float32)
    if relu:
      acc = jnp.maximum(acc, 0.0)
    o_ref[...] = acc

  return pl.pallas_call(
      body,
      grid=(npg // bm,),
      in_specs=[
          pl.BlockSpec((bm, d), lambda i: (i, 0)),
          pl.BlockSpec((r, bm, d), lambda i: (0, i, 0)),
          pl.BlockSpec((bm, r), lambda i: (i, 0)),
          pl.BlockSpec((r, d, d), lambda i: (0, 0, 0)),
          pl.BlockSpec((d, d), lambda i: (0, 0)),
          pl.BlockSpec((1, d), lambda i: (0, 0)),
      ],
      out_specs=pl.BlockSpec((bm, d), lambda i: (i, 0)),
      out_shape=jax.ShapeDtypeStruct((npg, d), jnp.float32),
  )


@functools.lru_cache(maxsize=None)
def _mha(b, npg, d, h, kb=512):
  hd = d // h
  nkv = npg // kb
  p16 = 16
  scale = 1.0 / (hd ** 0.5)

  def body(q_ref, x_ref, wq_ref, wk_ref, wv_ref, bq_ref, bk_ref, bv_ref,
           wo_ref, bo_ref, o_ref, qh_ref, s_ref, acc_ref):
    j = pl.program_id(1)

    @pl.when(j == 0)
    def _():
      qh_ref[...] = (jnp.dot(q_ref[...], wq_ref[...],
                             preferred_element_type=jnp.float32)
                     + bq_ref[...])

    @pl.when(j < nkv)
    def _scores():
      x = x_ref[0]
      k = jnp.dot(x, wk_ref[...], preferred_element_type=jnp.float32) \
          + bk_ref[...]
      jj = j
      for hh in range(h):
        qh = qh_ref[:, hh * hd:(hh + 1) * hd]
        kh = k[:, hh * hd:(hh + 1) * hd]
        s = lax.dot_general(qh, kh, (((1,), (1,)), ((), ())),
                            preferred_element_type=jnp.float32) * scale
        s_ref[hh, :, pl.ds(jj * kb, kb)] = s

    @pl.when(j == nkv)
    def _softmax():
      s = s_ref[...]
      m = jnp.max(s, axis=-1, keepdims=True)
      p = jnp.exp(s - m)
      s_ref[...] = p / jnp.sum(p, axis=-1, keepdims=True)
      acc_ref[...] = jnp.zeros_like(acc_ref)

    @pl.when(j >= nkv)
    def _values():
      x = x_ref[0]
      v = jnp.dot(x, wv_ref[...], preferred_element_type=jnp.float32) \
          + bv_ref[...]
      jj = j - nkv
      for hh in range(h):
        ph = s_ref[hh, :, pl.ds(jj * kb, kb)]
        vh = v[:, hh * hd:(hh + 1) * hd]
        acc_ref[hh] = acc_ref[hh] + jnp.dot(
            ph, vh, preferred_element_type=jnp.float32)

    @pl.when(j == 2 * nkv - 1)
    def _out():
      o = bo_ref[...] + jnp.zeros((p16, d), jnp.float32)
      for hh in range(h):
        o = o + jnp.dot(acc_ref[hh], wo_ref[pl.ds(hh * hd, hd)],
                        preferred_element_type=jnp.float32)
      o_ref[0] = o

  return pl.pallas_call(
      body,
      grid=(b, 2 * nkv),
      in_specs=[
          pl.BlockSpec((p16, d), lambda bb, j: (0, 0)),
          pl.BlockSpec((1, kb, d), lambda bb, j: (bb, j % nkv, 0)),
          pl.BlockSpec((d, d), lambda bb, j: (0, 0)),
          pl.BlockSpec((d, d), lambda bb, j: (0, 0)),
          pl.BlockSpec((d, d), lambda bb, j: (0, 0)),
          pl.BlockSpec((1, d), lambda bb, j: (0, 0)),
          pl.BlockSpec((1, d), lambda bb, j: (0, 0)),
          pl.BlockSpec((1, d), lambda bb, j: (0, 0)),
          pl.BlockSpec((d, d), lambda bb, j: (0, 0)),
          pl.BlockSpec((1, d), lambda bb, j: (0, 0)),
      ],
      out_specs=pl.BlockSpec((1, p16, d), lambda bb, j: (bb, 0, 0)),
      out_shape=jax.ShapeDtypeStruct((b, p16, d), jnp.float32),
      scratch_shapes=[
          pltpu.VMEM((p16, d), jnp.float32),
          pltpu.VMEM((h, p16, npg), jnp.float32),
          pltpu.VMEM((h, p16, hd), jnp.float32),
      ],
  )


def kernel(nodes, edges, types, table, W, root, bias, gprompt,
           in_proj_w, in_proj_b, out_proj_w, out_proj_b):
  b, npg = nodes.shape
  e = edges.shape[2]
  n_table, d = table.shape
  n_layers, r = W.shape[0], W.shape[1]
  p = gprompt.shape[0]
  h = 8
  chn = d // _LG

  nrows = b * npg
  idx3 = nodes.reshape(_NW, nrows // _NW // 64, 64)
  x0 = _gather_rows(n_table, nrows, d)(table, idx3).reshape(b, npg, d)

  def tile3(a):
    return a.reshape(_NS, e // _NS // 128, 128).astype(jnp.int32)

  embs = []
  for bb in range(b):
    src3 = tile3(edges[bb, 0])
    dst3 = tile3(edges[bb, 1])
    et3 = tile3(types[bb])
    x = x0[bb]
    cnt = None
    for l in range(n_layers):
      xv = x.reshape(npg * chn, _LG)
      if l == 0:
        sum3, cnt2 = _segsum(npg, e, r, d, True)(xv, src3, dst3, et3)
        cnt = cnt2[:, 0].reshape(r, npg).T
      else:
        sum3 = _segsum(npg, e, r, d, False)(xv, src3, dst3, et3)
      summed = sum3.reshape(r, npg, d)
      x = _combine(npg, d, r, l < n_layers - 1)(
          x, summed, cnt, W[l], root[l], bias[l][None])
    embs.append(x)

  node_embeddings = jnp.stack(embs, 0)

  gp16 = jnp.zeros((16, d), jnp.float32).at[:p].set(gprompt)
  wq_t = in_proj_w[:d].T
  wk_t = in_proj_w[d:2 * d].T
  wv_t = in_proj_w[2 * d:].T
  bq = in_proj_b[:d][None]
  bk = in_proj_b[d:2 * d][None]
  bv = in_proj_b[2 * d:][None]
  wo_t = out_proj_w.T
  bo = out_proj_b[None]
  mha_out = _mha(b, npg, d, h)(gp16, node_embeddings, wq_t, wk_t, wv_t,
                               bq, bk, bv, wo_t, bo)
  agg_embeddings = mha_out[:, :p, :]
  return (node_embeddings, agg_embeddings)


# speedup vs baseline: 3.9019x; 1.0025x over previous
import functools

import jax
import jax.numpy as jnp
from jax import lax
from jax.experimental import pallas as pl
from jax.experimental.pallas import tpu as pltpu
from jax.experimental.pallas import tpu_sc as plsc

_LG = 16
_NC = 2
_NS = 16
_NW = _NC * _NS


@functools.lru_cache(maxsize=None)
def _gather_rows(n_table, n_rows, d):
  ch = 64
  rpw = n_rows // _NW
  n_ch = rpw // ch
  mesh = plsc.VectorSubcoreMesh(core_axis_name="c", subcore_axis_name="s")

  @functools.partial(
      pl.kernel,
      out_type=jax.ShapeDtypeStruct((n_rows, d), jnp.float32),
      mesh=mesh,
      scratch_types=[
          pltpu.VMEM((n_ch, ch), jnp.int32),
          pltpu.VMEM((ch, d), jnp.float32),
          pltpu.SemaphoreType.DMA,
      ],
  )
  def k(table_hbm, idx_hbm, out_hbm, idx_v, rows_v, sem):
    cid = lax.axis_index("c")
    sid = lax.axis_index("s")
    wid = sid * _NC + cid
    base = wid * rpw
    pltpu.sync_copy(idx_hbm.at[wid], idx_v)

    def body(c, carry):
      pltpu.async_copy(table_hbm.at[idx_v.at[c]], rows_v, sem).wait()
      pltpu.sync_copy(rows_v, out_hbm.at[pl.ds(base + c * ch, ch)])
      return carry

    lax.fori_loop(0, n_ch, body, 0)

  return k


@functools.lru_cache(maxsize=None)
def _segsum(npg, e, r, d, with_cnt):
  chn = d // _LG
  nchc = chn // _NC
  ept = e // _NS
  g = ept // 128
  arows = r * npg
  trows = arows // _NS
  cnr = arows // _LG
  crt = cnr // _NS
  mesh = plsc.VectorSubcoreMesh(core_axis_name="c", subcore_axis_name="s")

  out_type = [jax.ShapeDtypeStruct((arows, chn, _LG), jnp.float32)]
  if with_cnt:
    out_type.append(jax.ShapeDtypeStruct((arows, _LG), jnp.float32))

  scratch = [
      pltpu.VMEM((g, 128), jnp.int32),
      pltpu.VMEM((g, 128), jnp.int32),
      pltpu.VMEM((g, 128), jnp.int32),
      pltpu.VMEM((16, 128, _LG), jnp.float32),
      pltpu.VMEM((512, _LG), jnp.float32),
      pltpu.SemaphoreType.DMA,
      pltpu.SemaphoreType.DMA,
      pltpu.VMEM_SHARED((arows, _LG), jnp.float32),
  ]

  def body(*refs):
    if with_cnt:
      (x_hbm, src_hbm, dst_hbm, et_hbm, sum_hbm, cnt_hbm,
       srcC, fdst, idxv, rows, zbuf, sem, sem2, acc) = refs
    else:
      (x_hbm, src_hbm, dst_hbm, et_hbm, sum_hbm,
       srcC, fdst, idxv, rows, zbuf, sem, sem2, acc) = refs

    cid = lax.axis_index("c")
    sid = lax.axis_index("s")

    pltpu.sync_copy(src_hbm.at[sid], srcC)
    pltpu.sync_copy(dst_hbm.at[sid], fdst)
    pltpu.sync_copy(et_hbm.at[sid], idxv)

    zv = jnp.zeros((_LG,), jnp.float32)

    def zfill(i, carry):
      zbuf[i, :] = zv
      return carry

    lax.fori_loop(0, 512, zfill, 0)

    def fixup(i, carry):
      j = i // 8
      kk = i % 8
      sl = pl.ds(kk * _LG, _LG)
      fdst[j, sl] = idxv[j, sl] * npg + fdst[j, sl]
      srcC[j, sl] = srcC[j, sl] * chn
      return carry

    lax.fori_loop(0, g * 8, fixup, 0)
    plsc.subcore_barrier()

    if with_cnt:
      @pl.when(cid == 0)
      def _count():
        for z in range(trows // 512):
          pltpu.sync_copy(zbuf, acc.at[pl.ds(sid * trows + z * 512, 512)])
        ones = jnp.ones((_LG,), jnp.float32)

        def ofill(i, carry):
          rows[0, i, :] = ones
          return carry

        lax.fori_loop(0, 128, ofill, 0)
        plsc.subcore_barrier()

        def cb(sb, carry):
          base = sb * 16
          ss = [pltpu.async_copy(rows.at[0], acc.at[fdst.at[base + j]],
                                 sem2, add=True) for j in range(16)]
          for dd in ss:
            dd.wait()
          return carry

        lax.fori_loop(0, g // 16, cb, 0)
        plsc.subcore_barrier()
        pltpu.sync_copy(acc.at[pl.ds(sid * trows, trows)],
                        cnt_hbm.at[pl.ds(sid * trows, trows)])

    def chunk(c, carry):
      cc = cid * nchc + c
      def ib(i, cy2):
        j = i // 8
        kk = i % 8
        sl = pl.ds(kk * _LG, _LG)
        idxv[j, sl] = srcC[j, sl] + cc
        return cy2

      lax.fori_loop(0, g * 8, ib, 0)
      for z in range(trows // 512):
        pltpu.sync_copy(zbuf, acc.at[pl.ds(sid * trows + z * 512, 512)])
      plsc.subcore_barrier()

      def sb_loop(sb, cy):
        base = sb * 16
        gs = [pltpu.async_copy(x_hbm.at[idxv.at[base + j]],
                               rows.at[j], sem) for j in range(16)]
        for dd in gs:
          dd.wait()
        ss = [pltpu.async_copy(rows.at[j], acc.at[fdst.at[base + j]],
                               sem2, add=True) for j in range(16)]
        for dd in ss:
          dd.wait()
        return cy

      lax.fori_loop(0, g // 16, sb_loop, 0)
      plsc.subcore_barrier()
      pltpu.sync_copy(acc.at[pl.ds(sid * trows, trows)],
                      sum_hbm.at[pl.ds(sid * trows, trows), cc])
      return carry

    lax.fori_loop(0, nchc, chunk, 0)

  return pl.kernel(
      body,
      out_type=tuple(out_type) if with_cnt else out_type[0],
      mesh=mesh,
      scratch_types=scratch,
      compiler_params=pltpu.CompilerParams(use_tc_tiling_on_sc=False),
  )


@functools.lru_cache(maxsize=None)
def _combine(npg, d, r, relu, bm=256):
  bf = jnp.bfloat16

  def body(x_ref, sum_ref, cnt_ref, w_ref, root_ref, bias_ref, o_ref):
    x = x_ref[...]
    inv = 1.0 / jnp.maximum(cnt_ref[...], 1.0)
    acc = jnp.dot(x.astype(bf), root_ref[...].astype(bf),
                  preferred_element_type=jnp.float32)
    acc = acc + bias_ref[...]
    for rr in range(r):
      m = sum_ref[rr] * inv[:, rr:rr + 1]
      acc = acc + jnp.dot(m.astype(bf), w_ref[rr].astype(bf),
                          preferred_element_type=jnp.float32)
    if relu:
      acc = jnp.maximum(acc, 0.0)
    o_ref[...] = acc

  return pl.pallas_call(
      body,
      grid=(npg // bm,),
      in_specs=[
          pl.BlockSpec((bm, d), lambda i: (i, 0)),
          pl.BlockSpec((r, bm, d), lambda i: (0, i, 0)),
          pl.BlockSpec((bm, r), lambda i: (i, 0)),
          pl.BlockSpec((r, d, d), lambda i: (0, 0, 0)),
          pl.BlockSpec((d, d), lambda i: (0, 0)),
          pl.BlockSpec((1, d), lambda i: (0, 0)),
      ],
      out_specs=pl.BlockSpec((bm, d), lambda i: (i, 0)),
      out_shape=jax.ShapeDtypeStruct((npg, d), jnp.float32),
  )


@functools.lru_cache(maxsize=None)
def _mha(b, npg, d, h, kb=512):
  hd = d // h
  nkv = npg // kb
  p16 = 16
  scale = 1.0 / (hd ** 0.5)

  bf = jnp.bfloat16

  def body(q_ref, x_ref, wq_ref, wk_ref, wv_ref, bq_ref, bk_ref, bv_ref,
           wo_ref, bo_ref, o_ref, qh_ref, s_ref, acc_ref):
    j = pl.program_id(1)

    @pl.when(j == 0)
    def _():
      qh_ref[...] = (jnp.dot(q_ref[...], wq_ref[...],
                             preferred_element_type=jnp.float32)
                     + bq_ref[...])

    @pl.when(j < nkv)
    def _scores():
      x = x_ref[0].astype(bf)
      k = jnp.dot(x, wk_ref[...].astype(bf),
                  preferred_element_type=jnp.float32) + bk_ref[...]
      jj = j
      for hh in range(h):
        qh = qh_ref[:, hh * hd:(hh + 1) * hd]
        kh = k[:, hh * hd:(hh + 1) * hd]
        s = lax.dot_general(qh, kh, (((1,), (1,)), ((), ())),
                            preferred_element_type=jnp.float32) * scale
        s_ref[hh, :, pl.ds(jj * kb, kb)] = s

    @pl.when(j == nkv)
    def _softmax():
      s = s_ref[...]
      m = jnp.max(s, axis=-1, keepdims=True)
      p = jnp.exp(s - m)
      s_ref[...] = p / jnp.sum(p, axis=-1, keepdims=True)
      acc_ref[...] = jnp.zeros_like(acc_ref)

    @pl.when(j >= nkv)
    def _values():
      x = x_ref[0].astype(bf)
      v = jnp.dot(x, wv_ref[...].astype(bf),
                  preferred_element_type=jnp.float32) + bv_ref[...]
      jj = j - nkv
      for hh in range(h):
        ph = s_ref[hh, :, pl.ds(jj * kb, kb)]
        vh = v[:, hh * hd:(hh + 1) * hd]
        acc_ref[hh] = acc_ref[hh] + jnp.dot(
            ph, vh, preferred_element_type=jnp.float32)

    @pl.when(j == 2 * nkv - 1)
    def _out():
      o = bo_ref[...] + jnp.zeros((p16, d), jnp.float32)
      for hh in range(h):
        o = o + jnp.dot(acc_ref[hh], wo_ref[pl.ds(hh * hd, hd)],
                        preferred_element_type=jnp.float32)
      o_ref[0] = o

  return pl.pallas_call(
      body,
      grid=(b, 2 * nkv),
      in_specs=[
          pl.BlockSpec((p16, d), lambda bb, j: (0, 0)),
          pl.BlockSpec((1, kb, d), lambda bb, j: (bb, j % nkv, 0)),
          pl.BlockSpec((d, d), lambda bb, j: (0, 0)),
          pl.BlockSpec((d, d), lambda bb, j: (0, 0)),
          pl.BlockSpec((d, d), lambda bb, j: (0, 0)),
          pl.BlockSpec((1, d), lambda bb, j: (0, 0)),
          pl.BlockSpec((1, d), lambda bb, j: (0, 0)),
          pl.BlockSpec((1, d), lambda bb, j: (0, 0)),
          pl.BlockSpec((d, d), lambda bb, j: (0, 0)),
          pl.BlockSpec((1, d), lambda bb, j: (0, 0)),
      ],
      out_specs=pl.BlockSpec((1, p16, d), lambda bb, j: (bb, 0, 0)),
      out_shape=jax.ShapeDtypeStruct((b, p16, d), jnp.float32),
      scratch_shapes=[
          pltpu.VMEM((p16, d), jnp.float32),
          pltpu.VMEM((h, p16, npg), jnp.float32),
          pltpu.VMEM((h, p16, hd), jnp.float32),
      ],
  )


def kernel(nodes, edges, types, table, W, root, bias, gprompt,
           in_proj_w, in_proj_b, out_proj_w, out_proj_b):
  b, npg = nodes.shape
  e = edges.shape[2]
  n_table, d = table.shape
  n_layers, r = W.shape[0], W.shape[1]
  p = gprompt.shape[0]
  h = 8
  chn = d // _LG

  nrows = b * npg
  idx3 = nodes.reshape(_NW, nrows // _NW // 64, 64)
  x0 = _gather_rows(n_table, nrows, d)(table, idx3).reshape(b, npg, d)

  def tile3(a):
    return a.reshape(_NS, e // _NS // 128, 128).astype(jnp.int32)

  embs = []
  for bb in range(b):
    src3 = tile3(edges[bb, 0])
    dst3 = tile3(edges[bb, 1])
    et3 = tile3(types[bb])
    x = x0[bb]
    cnt = None
    for l in range(n_layers):
      xv = x.reshape(npg * chn, _LG)
      if l == 0:
        sum3, cnt2 = _segsum(npg, e, r, d, True)(xv, src3, dst3, et3)
        cnt = cnt2[:, 0].reshape(r, npg).T
      else:
        sum3 = _segsum(npg, e, r, d, False)(xv, src3, dst3, et3)
      summed = sum3.reshape(r, npg, d)
      x = _combine(npg, d, r, l < n_layers - 1)(
          x, summed, cnt, W[l], root[l], bias[l][None])
    embs.append(x)

  node_embeddings = jnp.stack(embs, 0)

  gp16 = jnp.zeros((16, d), jnp.float32).at[:p].set(gprompt)
  wq_t = in_proj_w[:d].T
  wk_t = in_proj_w[d:2 * d].T
  wv_t = in_proj_w[2 * d:].T
  bq = in_proj_b[:d][None]
  bk = in_proj_b[d:2 * d][None]
  bv = in_proj_b[2 * d:][None]
  wo_t = out_proj_w.T
  bo = out_proj_b[None]
  mha_out = _mha(b, npg, d, h)(gp16, node_embeddings, wq_t, wk_t, wv_t,
                               bq, bk, bv, wo_t, bo)
  agg_embeddings = mha_out[:, :p, :]
  return (node_embeddings, agg_embeddings)


# speedup vs baseline: 3.9957x; 1.0240x over previous
import functools

import jax
import jax.numpy as jnp
from jax import lax
from jax.experimental import pallas as pl
from jax.experimental.pallas import tpu as pltpu
from jax.experimental.pallas import tpu_sc as plsc

_LG = 16
_NC = 2
_NS = 16
_NW = _NC * _NS


@functools.lru_cache(maxsize=None)
def _gather_rows(n_table, n_rows, d):
  ch = 64
  rpw = n_rows // _NW
  n_ch = rpw // ch
  mesh = plsc.VectorSubcoreMesh(core_axis_name="c", subcore_axis_name="s")

  @functools.partial(
      pl.kernel,
      out_type=jax.ShapeDtypeStruct((n_rows, d), jnp.float32),
      mesh=mesh,
      scratch_types=[
          pltpu.VMEM((n_ch, ch), jnp.int32),
          pltpu.VMEM((ch, d), jnp.float32),
          pltpu.SemaphoreType.DMA,
      ],
  )
  def k(table_hbm, idx_hbm, out_hbm, idx_v, rows_v, sem):
    cid = lax.axis_index("c")
    sid = lax.axis_index("s")
    wid = sid * _NC + cid
    base = wid * rpw
    pltpu.sync_copy(idx_hbm.at[wid], idx_v)

    def body(c, carry):
      pltpu.async_copy(table_hbm.at[idx_v.at[c]], rows_v, sem).wait()
      pltpu.sync_copy(rows_v, out_hbm.at[pl.ds(base + c * ch, ch)])
      return carry

    lax.fori_loop(0, n_ch, body, 0)

  return k


@functools.lru_cache(maxsize=None)
def _segsum(npg, e, r, d, with_cnt):
  chn = d // _LG
  nchc = chn // _NC
  ept = e // _NS
  g = ept // 128
  arows = r * npg
  trows = arows // _NS
  cnr = arows // _LG
  crt = cnr // _NS
  mesh = plsc.VectorSubcoreMesh(core_axis_name="c", subcore_axis_name="s")

  out_type = [jax.ShapeDtypeStruct((arows, chn, _LG), jnp.float32)]
  if with_cnt:
    out_type.append(jax.ShapeDtypeStruct((arows, _LG), jnp.float32))

  scratch = [
      pltpu.VMEM((g, 128), jnp.int32),
      pltpu.VMEM((g, 128), jnp.int32),
      pltpu.VMEM((g, 128), jnp.int32),
      pltpu.VMEM((16, 128, _LG), jnp.float32),
      pltpu.VMEM((512, _LG), jnp.float32),
      pltpu.SemaphoreType.DMA,
      pltpu.SemaphoreType.DMA,
      pltpu.VMEM_SHARED((arows, _LG), jnp.float32),
  ]

  def body(*refs):
    if with_cnt:
      (x_hbm, src_hbm, dst_hbm, et_hbm, sum_hbm, cnt_hbm,
       srcC, fdst, idxv, rows, zbuf, sem, sem2, acc) = refs
    else:
      (x_hbm, src_hbm, dst_hbm, et_hbm, sum_hbm,
       srcC, fdst, idxv, rows, zbuf, sem, sem2, acc) = refs

    cid = lax.axis_index("c")
    sid = lax.axis_index("s")

    pltpu.sync_copy(src_hbm.at[sid], srcC)
    pltpu.sync_copy(dst_hbm.at[sid], fdst)
    pltpu.sync_copy(et_hbm.at[sid], idxv)

    zv = jnp.zeros((_LG,), jnp.float32)

    def zfill(i, carry):
      zbuf[i, :] = zv
      return carry

    lax.fori_loop(0, 512, zfill, 0)

    def fixup(i, carry):
      j = i // 8
      kk = i % 8
      sl = pl.ds(kk * _LG, _LG)
      fdst[j, sl] = idxv[j, sl] * npg + fdst[j, sl]
      srcC[j, sl] = srcC[j, sl] * chn
      return carry

    lax.fori_loop(0, g * 8, fixup, 0)
    plsc.subcore_barrier()

    if with_cnt:
      @pl.when(cid == 0)
      def _count():
        for z in range(trows // 512):
          pltpu.sync_copy(zbuf, acc.at[pl.ds(sid * trows + z * 512, 512)])
        ones = jnp.ones((_LG,), jnp.float32)

        def ofill(i, carry):
          rows[0, i, :] = ones
          return carry

        lax.fori_loop(0, 128, ofill, 0)
        plsc.subcore_barrier()

        def cb(sb, carry):
          base = sb * 16
          ss = [pltpu.async_copy(rows.at[0], acc.at[fdst.at[base + j]],
                                 sem2, add=True) for j in range(16)]
          for dd in ss:
            dd.wait()
          return carry

        lax.fori_loop(0, g // 16, cb, 0)
        plsc.subcore_barrier()
        pltpu.sync_copy(acc.at[pl.ds(sid * trows, trows)],
                        cnt_hbm.at[pl.ds(sid * trows, trows)])

    def chunk(c, carry):
      cc = cid * nchc + c
      def ib(i, cy2):
        j = i // 8
        kk = i % 8
        sl = pl.ds(kk * _LG, _LG)
        idxv[j, sl] = srcC[j, sl] + cc
        return cy2

      lax.fori_loop(0, g * 8, ib, 0)
      for z in range(trows // 512):
        pltpu.sync_copy(zbuf, acc.at[pl.ds(sid * trows + z * 512, 512)])
      plsc.subcore_barrier()

      nsb = g // 8
      for j in range(8):
        pltpu.async_copy(x_hbm.at[idxv.at[j]], rows.at[j], sem)

      def sb_loop(sb, cy):
        base = sb * 8
        p = (sb % 2) * 8
        for j in range(8):
          pltpu.make_async_copy(x_hbm.at[idxv.at[base + j]],
                                rows.at[p + j], sem).wait()

        @pl.when(sb > 0)
        def _():
          b1 = (sb - 1) * 8
          p1 = ((sb - 1) % 2) * 8
          for j in range(8):
            pltpu.make_async_copy(rows.at[p1 + j],
                                  acc.at[fdst.at[b1 + j]], sem2).wait()

        for j in range(8):
          pltpu.async_copy(rows.at[p + j], acc.at[fdst.at[base + j]],
                           sem2, add=True)

        @pl.when(sb + 1 < nsb)
        def _():
          b2 = (sb + 1) * 8
          p2 = ((sb + 1) % 2) * 8
          for j in range(8):
            pltpu.async_copy(x_hbm.at[idxv.at[b2 + j]], rows.at[p2 + j], sem)

        return cy

      lax.fori_loop(0, nsb, sb_loop, 0)
      b1 = (nsb - 1) * 8
      p1 = ((nsb - 1) % 2) * 8
      for j in range(8):
        pltpu.make_async_copy(rows.at[p1 + j],
                              acc.at[fdst.at[b1 + j]], sem2).wait()
      plsc.subcore_barrier()
      pltpu.sync_copy(acc.at[pl.ds(sid * trows, trows)],
                      sum_hbm.at[pl.ds(sid * trows, trows), cc])
      return carry

    lax.fori_loop(0, nchc, chunk, 0)

  return pl.kernel(
      body,
      out_type=tuple(out_type) if with_cnt else out_type[0],
      mesh=mesh,
      scratch_types=scratch,
      compiler_params=pltpu.CompilerParams(use_tc_tiling_on_sc=False),
  )


@functools.lru_cache(maxsize=None)
def _combine(npg, d, r, relu, bm=256):
  bf = jnp.bfloat16

  def body(x_ref, sum_ref, cnt_ref, w_ref, root_ref, bias_ref, o_ref):
    x = x_ref[...]
    inv = 1.0 / jnp.maximum(cnt_ref[...], 1.0)
    acc = jnp.dot(x.astype(bf), root_ref[...].astype(bf),
                  preferred_element_type=jnp.float32)
    acc = acc + bias_ref[...]
    for rr in range(r):
      m = sum_ref[rr] * inv[:, rr:rr + 1]
      acc = acc + jnp.dot(m.astype(bf), w_ref[rr].astype(bf),
                          preferred_element_type=jnp.float32)
    if relu:
      acc = jnp.maximum(acc, 0.0)
    o_ref[...] = acc

  return pl.pallas_call(
      body,
      grid=(npg // bm,),
      in_specs=[
          pl.BlockSpec((bm, d), lambda i: (i, 0)),
          pl.BlockSpec((r, bm, d), lambda i: (0, i, 0)),
          pl.BlockSpec((bm, r), lambda i: (i, 0)),
          pl.BlockSpec((r, d, d), lambda i: (0, 0, 0)),
          pl.BlockSpec((d, d), lambda i: (0, 0)),
          pl.BlockSpec((1, d), lambda i: (0, 0)),
      ],
      out_specs=pl.BlockSpec((bm, d), lambda i: (i, 0)),
      out_shape=jax.ShapeDtypeStruct((npg, d), jnp.float32),
  )


@functools.lru_cache(maxsize=None)
def _mha(b, npg, d, h, kb=512):
  hd = d // h
  nkv = npg // kb
  p16 = 16
  scale = 1.0 / (hd ** 0.5)

  bf = jnp.bfloat16

  def body(q_ref, x_ref, wq_ref, wk_ref, wv_ref, bq_ref, bk_ref, bv_ref,
           wo_ref, bo_ref, o_ref, qh_ref, s_ref, acc_ref):
    j = pl.program_id(1)

    @pl.when(j == 0)
    def _():
      qh_ref[...] = (jnp.dot(q_ref[...], wq_ref[...],
                             preferred_element_type=jnp.float32)
                     + bq_ref[...])

    @pl.when(j < nkv)
    def _scores():
      x = x_ref[0].astype(bf)
      k = jnp.dot(x, wk_ref[...].astype(bf),
                  preferred_element_type=jnp.float32) + bk_ref[...]
      jj = j
      for hh in range(h):
        qh = qh_ref[:, hh * hd:(hh + 1) * hd]
        kh = k[:, hh * hd:(hh + 1) * hd]
        s = lax.dot_general(qh, kh, (((1,), (1,)), ((), ())),
                            preferred_element_type=jnp.float32) * scale
        s_ref[hh, :, pl.ds(jj * kb, kb)] = s

    @pl.when(j == nkv)
    def _softmax():
      s = s_ref[...]
      m = jnp.max(s, axis=-1, keepdims=True)
      p = jnp.exp(s - m)
      s_ref[...] = p / jnp.sum(p, axis=-1, keepdims=True)
      acc_ref[...] = jnp.zeros_like(acc_ref)

    @pl.when(j >= nkv)
    def _values():
      x = x_ref[0].astype(bf)
      v = jnp.dot(x, wv_ref[...].astype(bf),
                  preferred_element_type=jnp.float32) + bv_ref[...]
      jj = j - nkv
      for hh in range(h):
        ph = s_ref[hh, :, pl.ds(jj * kb, kb)]
        vh = v[:, hh * hd:(hh + 1) * hd]
        acc_ref[hh] = acc_ref[hh] + jnp.dot(
            ph, vh, preferred_element_type=jnp.float32)

    @pl.when(j == 2 * nkv - 1)
    def _out():
      o = bo_ref[...] + jnp.zeros((p16, d), jnp.float32)
      for hh in range(h):
        o = o + jnp.dot(acc_ref[hh], wo_ref[pl.ds(hh * hd, hd)],
                        preferred_element_type=jnp.float32)
      o_ref[0] = o

  return pl.pallas_call(
      body,
      grid=(b, 2 * nkv),
      in_specs=[
          pl.BlockSpec((p16, d), lambda bb, j: (0, 0)),
          pl.BlockSpec((1, kb, d), lambda bb, j: (bb, j % nkv, 0)),
          pl.BlockSpec((d, d), lambda bb, j: (0, 0)),
          pl.BlockSpec((d, d), lambda bb, j: (0, 0)),
          pl.BlockSpec((d, d), lambda bb, j: (0, 0)),
          pl.BlockSpec((1, d), lambda bb, j: (0, 0)),
          pl.BlockSpec((1, d), lambda bb, j: (0, 0)),
          pl.BlockSpec((1, d), lambda bb, j: (0, 0)),
          pl.BlockSpec((d, d), lambda bb, j: (0, 0)),
          pl.BlockSpec((1, d), lambda bb, j: (0, 0)),
      ],
      out_specs=pl.BlockSpec((1, p16, d), lambda bb, j: (bb, 0, 0)),
      out_shape=jax.ShapeDtypeStruct((b, p16, d), jnp.float32),
      scratch_shapes=[
          pltpu.VMEM((p16, d), jnp.float32),
          pltpu.VMEM((h, p16, npg), jnp.float32),
          pltpu.VMEM((h, p16, hd), jnp.float32),
      ],
  )


def kernel(nodes, edges, types, table, W, root, bias, gprompt,
           in_proj_w, in_proj_b, out_proj_w, out_proj_b):
  b, npg = nodes.shape
  e = edges.shape[2]
  n_table, d = table.shape
  n_layers, r = W.shape[0], W.shape[1]
  p = gprompt.shape[0]
  h = 8
  chn = d // _LG

  nrows = b * npg
  idx3 = nodes.reshape(_NW, nrows // _NW // 64, 64)
  x0 = _gather_rows(n_table, nrows, d)(table, idx3).reshape(b, npg, d)

  def tile3(a):
    return a.reshape(_NS, e // _NS // 128, 128).astype(jnp.int32)

  embs = []
  for bb in range(b):
    src3 = tile3(edges[bb, 0])
    dst3 = tile3(edges[bb, 1])
    et3 = tile3(types[bb])
    x = x0[bb]
    cnt = None
    for l in range(n_layers):
      xv = x.reshape(npg * chn, _LG)
      if l == 0:
        sum3, cnt2 = _segsum(npg, e, r, d, True)(xv, src3, dst3, et3)
        cnt = cnt2[:, 0].reshape(r, npg).T
      else:
        sum3 = _segsum(npg, e, r, d, False)(xv, src3, dst3, et3)
      summed = sum3.reshape(r, npg, d)
      x = _combine(npg, d, r, l < n_layers - 1)(
          x, summed, cnt, W[l], root[l], bias[l][None])
    embs.append(x)

  node_embeddings = jnp.stack(embs, 0)

  gp16 = jnp.zeros((16, d), jnp.float32).at[:p].set(gprompt)
  wq_t = in_proj_w[:d].T
  wk_t = in_proj_w[d:2 * d].T
  wv_t = in_proj_w[2 * d:].T
  bq = in_proj_b[:d][None]
  bk = in_proj_b[d:2 * d][None]
  bv = in_proj_b[2 * d:][None]
  wo_t = out_proj_w.T
  bo = out_proj_b[None]
  mha_out = _mha(b, npg, d, h)(gp16, node_embeddings, wq_t, wk_t, wv_t,
                               bq, bk, bv, wo_t, bo)
  agg_embeddings = mha_out[:, :p, :]
  return (node_embeddings, agg_embeddings)
